# Initial kernel scaffold; baseline (speedup 1.0000x reference)
import jax
import jax.numpy as jnp
from jax.experimental import pallas as pl


def kernel(h, edge_index, W, b):
    raise NotImplementedError("write your pallas kernel here")


# speedup vs baseline: 21.6677x; 21.6677x over previous
import functools

import jax
import jax.numpy as jnp
from jax import lax
from jax.experimental import pallas as pl
from jax.experimental.pallas import tpu as pltpu
from jax.experimental.pallas import tpu_sc as plsc

N = 10000
NPAD = 10240
E = 320000
E_PAD = 323584
F_IN = 128
C_OUT = 7
C8 = 8

ROWS = E_PAD // 128
ROWS_PER_TILE_DEG = ROWS // 16
ROWS_PER_W = ROWS // 32
NODES_PER_TILE = NPAD // 16

_mesh = plsc.VectorSubcoreMesh(core_axis_name="c", subcore_axis_name="s")
_sc_params = pltpu.CompilerParams(use_tc_tiling_on_sc=False)


def _deg_body(edges_hbm, zeros_hbm, deg_hbm, idx_v, ones_v, out_v, shared):
    c = lax.axis_index("c")
    s = lax.axis_index("s")
    for i in range(8):
        ones_v[pl.ds(i * 16, 16)] = jnp.ones((16,), jnp.float32)
    nslc = pl.ds(s * NODES_PER_TILE, NODES_PER_TILE)
    pltpu.sync_copy(zeros_hbm.at[nslc], shared.at[nslc])
    for j in range(2):
        pltpu.sync_copy(edges_hbm.at[c, s * 2 + j], idx_v.at[j])
    plsc.subcore_barrier()

    def scat(b, carry):
        for j in range(2):
            pltpu.sync_copy(ones_v, shared.at[idx_v.at[j, b]], add=True)
        return carry

    lax.fori_loop(0, ROWS_PER_W, scat, 0)
    plsc.subcore_barrier()
    pltpu.sync_copy(shared.at[nslc], out_v)
    pltpu.sync_copy(out_v, deg_hbm.at[pl.ds(c * NPAD + s * NODES_PER_TILE,
                                            NODES_PER_TILE)])


_deg_call = functools.partial(
    pl.kernel,
    out_type=jax.ShapeDtypeStruct((2 * NPAD,), jnp.float32),
    mesh=_mesh,
    compiler_params=_sc_params,
    scratch_types=[
        pltpu.VMEM((2, ROWS_PER_W, 128), jnp.int32),
        pltpu.VMEM((128,), jnp.float32),
        pltpu.VMEM((NODES_PER_TILE,), jnp.float32),
        pltpu.VMEM_SHARED((NPAD,), jnp.float32),
    ],
)(_deg_body)


def _agg_body(edges_hbm, feat_hbm, zeros_hbm, rst_hbm,
              src_v, dst_v, msgs_v, out_v, shared, feat_sh, gsem):
    c = lax.axis_index("c")
    s = lax.axis_index("s")
    w = s * 2 + c
    pltpu.sync_copy(edges_hbm.at[0, w], src_v)
    pltpu.sync_copy(edges_hbm.at[1, w], dst_v)
    nslc = pl.ds(s * NODES_PER_TILE, NODES_PER_TILE)
    pltpu.sync_copy(zeros_hbm.at[nslc], shared.at[nslc])
    pltpu.sync_copy(feat_hbm.at[nslc], feat_sh.at[nslc])
    plsc.subcore_barrier()

    def g_start(b, carry):
        pltpu.async_copy(feat_sh.at[src_v.at[b]], msgs_v.at[b], gsem)
        return carry

    lax.fori_loop(0, ROWS_PER_W, g_start, 0)

    def g_wait(b, carry):
        pltpu.make_async_copy(feat_sh.at[src_v.at[b]], msgs_v.at[b], gsem).wait()
        return carry

    lax.fori_loop(0, ROWS_PER_W, g_wait, 0)

    def scat(b, carry):
        pltpu.sync_copy(msgs_v.at[b], shared.at[dst_v.at[b]], add=True)
        return carry

    lax.fori_loop(0, ROWS_PER_W, scat, 0)
    plsc.subcore_barrier()
    pltpu.sync_copy(shared.at[nslc], out_v)
    pltpu.sync_copy(out_v, rst_hbm.at[c, nslc])


_agg_call = functools.partial(
    pl.kernel,
    out_type=jax.ShapeDtypeStruct((2, NPAD, C8), jnp.float32),
    mesh=_mesh,
    compiler_params=_sc_params,
    scratch_types=[
        pltpu.VMEM((ROWS_PER_W, 128), jnp.int32),
        pltpu.VMEM((ROWS_PER_W, 128), jnp.int32),
        pltpu.VMEM((ROWS_PER_W, 128, C8), jnp.float32),
        pltpu.VMEM((NODES_PER_TILE, C8), jnp.float32),
        pltpu.VMEM_SHARED((NPAD, C8), jnp.float32),
        pltpu.VMEM_SHARED((NPAD, C8), jnp.float32),
        pltpu.SemaphoreType.DMA,
    ],
)(_agg_body)


def _mm_body(h_ref, w_ref, od_ref, id_ref, feat_ref, ndst_ref):
    y = jnp.dot(h_ref[...], w_ref[...], preferred_element_type=jnp.float32)
    nsrc = lax.rsqrt(jnp.maximum(od_ref[...], 1.0))
    feat_ref[...] = y * nsrc
    ndst = lax.rsqrt(jnp.maximum(id_ref[...], 1.0))
    ndst_ref[...] = jnp.broadcast_to(ndst, (NPAD, C8))


def _fin_body(rst_ref, ndst_ref, b_ref, out_ref):
    out_ref[...] = (rst_ref[0] + rst_ref[1]) * ndst_ref[...] + b_ref[...]


def kernel(h, edge_index, W, b):
    e32 = edge_index.astype(jnp.int32)
    n_extra = E_PAD - E
    pad_idx = N + (jnp.arange(n_extra, dtype=jnp.int32) % (NPAD - N))
    edges4 = jnp.concatenate(
        [e32, jnp.stack([pad_idx, pad_idx])], axis=1).reshape(2, 32, ROWS_PER_W, 128)
    h_pad = jnp.pad(h, ((0, NPAD - N), (0, 0)))
    W8 = jnp.pad(W, ((0, 0), (0, C8 - C_OUT)))
    b8 = jnp.pad(b, (0, C8 - C_OUT)).reshape(1, C8)
    zeros1 = jnp.zeros((NPAD,), jnp.float32)
    zeros8 = jnp.zeros((NPAD, C8), jnp.float32)

    degs = _deg_call(edges4, zeros1).reshape(2, NPAD)
    od = degs[0].reshape(NPAD, 1)
    idg = degs[1].reshape(NPAD, 1)

    feat, ndst8 = pl.pallas_call(
        _mm_body,
        out_shape=[jax.ShapeDtypeStruct((NPAD, C8), jnp.float32),
                   jax.ShapeDtypeStruct((NPAD, C8), jnp.float32)],
    )(h_pad, W8, od, idg)

    rst = _agg_call(edges4, feat, zeros8)

    out8 = pl.pallas_call(
        _fin_body,
        out_shape=jax.ShapeDtypeStruct((NPAD, C8), jnp.float32),
    )(rst, ndst8, b8)
    return out8[:N, :C_OUT]


# speedup vs baseline: 24.5880x; 1.1348x over previous
import functools

import jax
import jax.numpy as jnp
from jax import lax
from jax.experimental import pallas as pl
from jax.experimental.pallas import tpu as pltpu
from jax.experimental.pallas import tpu_sc as plsc

N = 10000
NPAD = 10240
E = 320000
E_PAD = 323584
F_IN = 128
C_OUT = 7
C8 = 8

ROWS_PER_W = 79
NODES_PER_TILE = NPAD // 16

_mesh = plsc.VectorSubcoreMesh(core_axis_name="c", subcore_axis_name="s")
_sc_params = pltpu.CompilerParams(use_tc_tiling_on_sc=False)


def _deg_body(edges_hbm, zeros_hbm, deg_hbm, idx_v, ones_v, out_v, shared, ssem):
    c = lax.axis_index("c")
    s = lax.axis_index("s")
    for i in range(8):
        ones_v[pl.ds(i * 16, 16)] = jnp.ones((16,), jnp.float32)
    nslc = pl.ds(s * NODES_PER_TILE, NODES_PER_TILE)
    pltpu.sync_copy(zeros_hbm.at[nslc], shared.at[nslc])
    for j in range(2):
        pltpu.sync_copy(edges_hbm.at[c, s * 2 + j], idx_v.at[j])
    plsc.subcore_barrier()

    def scat(b, carry):
        for j in range(2):
            pltpu.async_copy(ones_v, shared.at[idx_v.at[j, b]], ssem, add=True)
        return carry

    lax.fori_loop(0, ROWS_PER_W, scat, 0)

    def drain(b, carry):
        for j in range(2):
            pltpu.make_async_copy(ones_v, shared.at[idx_v.at[j, b]], ssem).wait()
        return carry

    lax.fori_loop(0, ROWS_PER_W, drain, 0)
    plsc.subcore_barrier()
    pltpu.sync_copy(shared.at[nslc], out_v)
    pltpu.sync_copy(out_v, deg_hbm.at[pl.ds(c * NPAD + s * NODES_PER_TILE,
                                            NODES_PER_TILE)])


_deg_call = functools.partial(
    pl.kernel,
    out_type=jax.ShapeDtypeStruct((2 * NPAD,), jnp.float32),
    mesh=_mesh,
    compiler_params=_sc_params,
    scratch_types=[
        pltpu.VMEM((2, ROWS_PER_W, 128), jnp.int32),
        pltpu.VMEM((128,), jnp.float32),
        pltpu.VMEM((NODES_PER_TILE,), jnp.float32),
        pltpu.VMEM_SHARED((NPAD,), jnp.float32),
        pltpu.SemaphoreType.DMA,
    ],
)(_deg_body)


def _agg_body(edges_hbm, feat_hbm, zeros_hbm, rst_hbm,
              src_v, dst_v, msgs_v, out_v, shared, feat_sh, gsem):
    c = lax.axis_index("c")
    s = lax.axis_index("s")
    w = s * 2 + c
    pltpu.sync_copy(edges_hbm.at[0, w], src_v)
    pltpu.sync_copy(edges_hbm.at[1, w], dst_v)
    nslc = pl.ds(s * NODES_PER_TILE, NODES_PER_TILE)
    pltpu.sync_copy(zeros_hbm.at[nslc], shared.at[nslc])
    pltpu.sync_copy(feat_hbm.at[nslc], feat_sh.at[nslc])
    plsc.subcore_barrier()

    def g_start(b, carry):
        pltpu.async_copy(feat_sh.at[src_v.at[b]], msgs_v.at[b], gsem)
        return carry

    lax.fori_loop(0, ROWS_PER_W, g_start, 0)

    def g_wait(b, carry):
        pltpu.make_async_copy(feat_sh.at[src_v.at[b]], msgs_v.at[b], gsem).wait()
        return carry

    lax.fori_loop(0, ROWS_PER_W, g_wait, 0)

    def scat(b, carry):
        pltpu.async_copy(msgs_v.at[b], shared.at[dst_v.at[b]], gsem, add=True)
        return carry

    lax.fori_loop(0, ROWS_PER_W, scat, 0)

    def s_wait(b, carry):
        pltpu.make_async_copy(msgs_v.at[b], shared.at[dst_v.at[b]], gsem).wait()
        return carry

    lax.fori_loop(0, ROWS_PER_W, s_wait, 0)
    plsc.subcore_barrier()
    pltpu.sync_copy(shared.at[nslc], out_v)
    pltpu.sync_copy(out_v, rst_hbm.at[c, nslc])


_agg_call = functools.partial(
    pl.kernel,
    out_type=jax.ShapeDtypeStruct((2, NPAD, C8), jnp.float32),
    mesh=_mesh,
    compiler_params=_sc_params,
    scratch_types=[
        pltpu.VMEM((ROWS_PER_W, 128), jnp.int32),
        pltpu.VMEM((ROWS_PER_W, 128), jnp.int32),
        pltpu.VMEM((ROWS_PER_W, 128, C8), jnp.float32),
        pltpu.VMEM((NODES_PER_TILE, C8), jnp.float32),
        pltpu.VMEM_SHARED((NPAD, C8), jnp.float32),
        pltpu.VMEM_SHARED((NPAD, C8), jnp.float32),
        pltpu.SemaphoreType.DMA,
    ],
)(_agg_body)


_MM_BLK = 1024


def _mm_body(h_ref, w_ref, y_ref):
    y_ref[...] = jnp.dot(h_ref[...], w_ref[...],
                         preferred_element_type=jnp.float32)


def _norm_body(y_ref, od_ref, id_ref, feat_ref, ndst_ref):
    nsrc = lax.rsqrt(jnp.maximum(od_ref[...], 1.0))
    feat_ref[...] = y_ref[...] * nsrc
    ndst = lax.rsqrt(jnp.maximum(id_ref[...], 1.0))
    ndst_ref[...] = jnp.broadcast_to(ndst, (_MM_BLK, C8))


_FIN_BLK = 1000


def _fin_body(rst_ref, ndst_ref, b_ref, out_ref):
    acc = (rst_ref[0] + rst_ref[1]) * ndst_ref[...] + b_ref[...]
    out_ref[...] = acc[:, :C_OUT]


def kernel(h, edge_index, W, b):
    e32 = edge_index.astype(jnp.int32)
    n_extra = E_PAD - E
    pad_idx = N + (jnp.arange(n_extra, dtype=jnp.int32) % (NPAD - N))
    edges4 = jnp.concatenate(
        [e32, jnp.stack([pad_idx, pad_idx])], axis=1).reshape(2, 32, ROWS_PER_W, 128)
    W8 = jnp.pad(W, ((0, 0), (0, C8 - C_OUT)))
    b8 = jnp.pad(b, (0, C8 - C_OUT)).reshape(1, C8)
    zeros1 = jnp.zeros((NPAD,), jnp.float32)
    zeros8 = jnp.zeros((NPAD, C8), jnp.float32)

    degs = _deg_call(edges4, zeros1)
    od = degs[:NPAD].reshape(NPAD, 1)
    idg = degs[NPAD:].reshape(NPAD, 1)

    y = pl.pallas_call(
        _mm_body,
        grid=(NPAD // _MM_BLK,),
        in_specs=[pl.BlockSpec((_MM_BLK, F_IN), lambda i: (i, 0)),
                  pl.BlockSpec((F_IN, C8), lambda i: (0, 0))],
        out_specs=pl.BlockSpec((_MM_BLK, C8), lambda i: (i, 0)),
        out_shape=jax.ShapeDtypeStruct((NPAD, C8), jnp.float32),
    )(h, W8)

    feat, ndst8 = pl.pallas_call(
        _norm_body,
        grid=(NPAD // _MM_BLK,),
        in_specs=[pl.BlockSpec((_MM_BLK, C8), lambda i: (i, 0)),
                  pl.BlockSpec((_MM_BLK, 1), lambda i: (i, 0)),
                  pl.BlockSpec((_MM_BLK, 1), lambda i: (i, 0))],
        out_specs=[pl.BlockSpec((_MM_BLK, C8), lambda i: (i, 0)),
                   pl.BlockSpec((_MM_BLK, C8), lambda i: (i, 0))],
        out_shape=[jax.ShapeDtypeStruct((NPAD, C8), jnp.float32),
                   jax.ShapeDtypeStruct((NPAD, C8), jnp.float32)],
    )(y, od, idg)

    rst = _agg_call(edges4, feat, zeros8)

    out = pl.pallas_call(
        _fin_body,
        grid=(N // _FIN_BLK,),
        in_specs=[pl.BlockSpec((2, _FIN_BLK, C8), lambda i: (0, i, 0)),
                  pl.BlockSpec((_FIN_BLK, C8), lambda i: (i, 0)),
                  pl.BlockSpec((1, C8), lambda i: (0, 0))],
        out_specs=pl.BlockSpec((_FIN_BLK, C_OUT), lambda i: (i, 0)),
        out_shape=jax.ShapeDtypeStruct((N, C_OUT), jnp.float32),
    )(rst, ndst8, b8)
    return out


# speedup vs baseline: 25.5849x; 1.0405x over previous
import functools

import jax
import jax.numpy as jnp
from jax import lax
from jax.experimental import pallas as pl
from jax.experimental.pallas import tpu as pltpu
from jax.experimental.pallas import tpu_sc as plsc

N = 10000
NPAD = 10240
E = 320000
E_PAD = 323584
F_IN = 128
C_OUT = 7
C8 = 8

ROWS_PER_W = 79
NODES_PER_TILE = NPAD // 16

_mesh = plsc.VectorSubcoreMesh(core_axis_name="c", subcore_axis_name="s")
_sc_params = pltpu.CompilerParams(use_tc_tiling_on_sc=False)


def _deg_body(edges_hbm, zeros_hbm, deg_hbm, idx_v, ones_v, out_v, shared, ssem):
    c = lax.axis_index("c")
    s = lax.axis_index("s")
    for i in range(8):
        ones_v[pl.ds(i * 16, 16)] = jnp.ones((16,), jnp.float32)
    nslc = pl.ds(s * NODES_PER_TILE, NODES_PER_TILE)
    pltpu.sync_copy(zeros_hbm.at[nslc], shared.at[nslc])
    for j in range(2):
        pltpu.sync_copy(edges_hbm.at[c, s * 2 + j], idx_v.at[j])
    plsc.subcore_barrier()

    def scat(b, carry):
        for j in range(2):
            pltpu.async_copy(ones_v, shared.at[idx_v.at[j, b]], ssem, add=True)
        return carry

    lax.fori_loop(0, ROWS_PER_W, scat, 0)

    def drain(b, carry):
        for j in range(2):
            pltpu.make_async_copy(ones_v, shared.at[idx_v.at[j, b]], ssem).wait()
        return carry

    lax.fori_loop(0, ROWS_PER_W, drain, 0)
    plsc.subcore_barrier()
    pltpu.sync_copy(shared.at[nslc], out_v)
    pltpu.sync_copy(out_v, deg_hbm.at[pl.ds(c * NPAD + s * NODES_PER_TILE,
                                            NODES_PER_TILE)])


_deg_call = functools.partial(
    pl.kernel,
    out_type=jax.ShapeDtypeStruct((2 * NPAD,), jnp.float32),
    mesh=_mesh,
    compiler_params=_sc_params,
    scratch_types=[
        pltpu.VMEM((2, ROWS_PER_W, 128), jnp.int32),
        pltpu.VMEM((128,), jnp.float32),
        pltpu.VMEM((NODES_PER_TILE,), jnp.float32),
        pltpu.VMEM_SHARED((NPAD,), jnp.float32),
        pltpu.SemaphoreType.DMA,
    ],
)(_deg_body)


def _agg_body(edges_hbm, feat_hbm, zeros_hbm, rst_hbm,
              src_v, dst_v, msgs_v, out_v, shared, feat_sh, gsem):
    c = lax.axis_index("c")
    s = lax.axis_index("s")
    w = s * 2 + c
    pltpu.sync_copy(edges_hbm.at[0, w], src_v)
    pltpu.sync_copy(edges_hbm.at[1, w], dst_v)
    nslc = pl.ds(s * NODES_PER_TILE, NODES_PER_TILE)
    pltpu.sync_copy(zeros_hbm.at[nslc], shared.at[nslc])
    pltpu.sync_copy(feat_hbm.at[nslc], feat_sh.at[nslc])
    plsc.subcore_barrier()

    def g_start(b, carry):
        pltpu.async_copy(feat_sh.at[src_v.at[b]], msgs_v.at[b], gsem)
        return carry

    lax.fori_loop(0, ROWS_PER_W, g_start, 0)

    def g_wait(b, carry):
        pltpu.make_async_copy(feat_sh.at[src_v.at[b]], msgs_v.at[b], gsem).wait()
        return carry

    lax.fori_loop(0, ROWS_PER_W, g_wait, 0)

    def scat(b, carry):
        pltpu.async_copy(msgs_v.at[b], shared.at[dst_v.at[b]], gsem, add=True)
        return carry

    lax.fori_loop(0, ROWS_PER_W, scat, 0)

    def s_wait(b, carry):
        pltpu.make_async_copy(msgs_v.at[b], shared.at[dst_v.at[b]], gsem).wait()
        return carry

    lax.fori_loop(0, ROWS_PER_W, s_wait, 0)
    plsc.subcore_barrier()
    pltpu.sync_copy(shared.at[nslc], out_v)
    pltpu.sync_copy(out_v, rst_hbm.at[c, nslc])


_agg_call = functools.partial(
    pl.kernel,
    out_type=jax.ShapeDtypeStruct((2, NPAD, C8), jnp.float32),
    mesh=_mesh,
    compiler_params=_sc_params,
    scratch_types=[
        pltpu.VMEM((ROWS_PER_W, 128), jnp.int32),
        pltpu.VMEM((ROWS_PER_W, 128), jnp.int32),
        pltpu.VMEM((ROWS_PER_W, 128, C8), jnp.float32),
        pltpu.VMEM((NODES_PER_TILE, C8), jnp.float32),
        pltpu.VMEM_SHARED((NPAD, C8), jnp.float32),
        pltpu.VMEM_SHARED((NPAD, C8), jnp.float32),
        pltpu.SemaphoreType.DMA,
    ],
)(_agg_body)


_MM_BLK = 1024


def _mm_body(h_ref, w_ref, y_ref):
    y_ref[...] = jnp.dot(h_ref[...], w_ref[...],
                         preferred_element_type=jnp.float32)


def _norm_body(y_ref, od_ref, feat_ref):
    nsrc = lax.rsqrt(jnp.maximum(od_ref[...], 1.0))
    feat_ref[...] = y_ref[...] * nsrc


_FIN_BLK = 1000


def _fin_body(rst_ref, id_ref, b_ref, out_ref):
    ndst = lax.rsqrt(jnp.maximum(id_ref[...], 1.0))
    acc = (rst_ref[0] + rst_ref[1]) * ndst + b_ref[...]
    out_ref[...] = acc[:, :C_OUT]


def kernel(h, edge_index, W, b):
    e32 = edge_index.astype(jnp.int32)
    n_extra = E_PAD - E
    pad_idx = N + (jnp.arange(n_extra, dtype=jnp.int32) % (NPAD - N))
    edges4 = jnp.concatenate(
        [e32, jnp.stack([pad_idx, pad_idx])], axis=1).reshape(2, 32, ROWS_PER_W, 128)
    W8 = jnp.pad(W, ((0, 0), (0, C8 - C_OUT)))
    b8 = jnp.pad(b, (0, C8 - C_OUT)).reshape(1, C8)
    zeros1 = jnp.zeros((NPAD,), jnp.float32)
    zeros8 = jnp.zeros((NPAD, C8), jnp.float32)

    degs = _deg_call(edges4, zeros1)
    od = degs[:NPAD].reshape(NPAD, 1)
    idg = degs[NPAD:].reshape(NPAD, 1)

    y = pl.pallas_call(
        _mm_body,
        grid=(NPAD // _MM_BLK,),
        in_specs=[pl.BlockSpec((_MM_BLK, F_IN), lambda i: (i, 0)),
                  pl.BlockSpec((F_IN, C8), lambda i: (0, 0))],
        out_specs=pl.BlockSpec((_MM_BLK, C8), lambda i: (i, 0)),
        out_shape=jax.ShapeDtypeStruct((NPAD, C8), jnp.float32),
    )(h, W8)

    feat = pl.pallas_call(
        _norm_body,
        grid=(NPAD // _MM_BLK,),
        in_specs=[pl.BlockSpec((_MM_BLK, C8), lambda i: (i, 0)),
                  pl.BlockSpec((_MM_BLK, 1), lambda i: (i, 0))],
        out_specs=pl.BlockSpec((_MM_BLK, C8), lambda i: (i, 0)),
        out_shape=jax.ShapeDtypeStruct((NPAD, C8), jnp.float32),
    )(y, od)

    rst = _agg_call(edges4, feat, zeros8)

    out = pl.pallas_call(
        _fin_body,
        grid=(N // _FIN_BLK,),
        in_specs=[pl.BlockSpec((2, _FIN_BLK, C8), lambda i: (0, i, 0)),
                  pl.BlockSpec((_FIN_BLK, 1), lambda i: (i, 0)),
                  pl.BlockSpec((1, C8), lambda i: (0, 0))],
        out_specs=pl.BlockSpec((_FIN_BLK, C_OUT), lambda i: (i, 0)),
        out_shape=jax.ShapeDtypeStruct((N, C_OUT), jnp.float32),
    )(rst, idg, b8)
    return out


# speedup vs baseline: 26.6866x; 1.0431x over previous
import functools

import jax
import jax.numpy as jnp
from jax import lax
from jax.experimental import pallas as pl
from jax.experimental.pallas import tpu as pltpu
from jax.experimental.pallas import tpu_sc as plsc

N = 10000
NPAD = 10240
E = 320000
F_IN = 128
C_OUT = 7
C8 = 8

E_PER_TILE = E // 16
NB_DEG = E_PER_TILE // 128
TAIL_DEG = E_PER_TILE - NB_DEG * 128
E_PER_W = E // 32
NB_AGG = E_PER_W // 128
TAIL_AGG = E_PER_W - NB_AGG * 128
NODES_PER_TILE = NPAD // 16

_mesh = plsc.VectorSubcoreMesh(core_axis_name="c", subcore_axis_name="s")
_sc_params = pltpu.CompilerParams(use_tc_tiling_on_sc=False)


def _deg_body(edges_hbm, zeros_hbm, deg_hbm, idx_v, ones_v, out_v, shared, ssem):
    c = lax.axis_index("c")
    s = lax.axis_index("s")
    for i in range(8):
        ones_v[pl.ds(i * 16, 16)] = jnp.ones((16,), jnp.float32)
    nslc = pl.ds(s * NODES_PER_TILE, NODES_PER_TILE)
    pltpu.sync_copy(zeros_hbm.at[nslc], shared.at[nslc])
    pltpu.sync_copy(edges_hbm.at[pl.ds(c * E + s * E_PER_TILE, E_PER_TILE)], idx_v)
    plsc.subcore_barrier()

    def scat(b, carry):
        off = pl.multiple_of(b * 128, 128)
        pltpu.async_copy(ones_v, shared.at[idx_v.at[pl.ds(off, 128)]], ssem,
                         add=True)
        return carry

    lax.fori_loop(0, NB_DEG, scat, 0)
    pltpu.async_copy(ones_v.at[pl.ds(0, TAIL_DEG)],
                     shared.at[idx_v.at[pl.ds(NB_DEG * 128, TAIL_DEG)]], ssem,
                     add=True)

    def drain(b, carry):
        off = pl.multiple_of(b * 128, 128)
        pltpu.make_async_copy(ones_v, shared.at[idx_v.at[pl.ds(off, 128)]],
                              ssem).wait()
        return carry

    lax.fori_loop(0, NB_DEG, drain, 0)
    pltpu.make_async_copy(ones_v.at[pl.ds(0, TAIL_DEG)],
                          shared.at[idx_v.at[pl.ds(NB_DEG * 128, TAIL_DEG)]],
                          ssem).wait()
    plsc.subcore_barrier()
    pltpu.sync_copy(shared.at[nslc], out_v)
    pltpu.sync_copy(out_v, deg_hbm.at[pl.ds(c * NPAD + s * NODES_PER_TILE,
                                            NODES_PER_TILE)])


_deg_call = functools.partial(
    pl.kernel,
    out_type=jax.ShapeDtypeStruct((2 * NPAD,), jnp.float32),
    mesh=_mesh,
    compiler_params=_sc_params,
    scratch_types=[
        pltpu.VMEM((E_PER_TILE,), jnp.int32),
        pltpu.VMEM((128,), jnp.float32),
        pltpu.VMEM((NODES_PER_TILE,), jnp.float32),
        pltpu.VMEM_SHARED((NPAD,), jnp.float32),
        pltpu.SemaphoreType.DMA,
    ],
)(_deg_body)


def _agg_body(edges_hbm, feat_hbm, zeros_hbm, rst_hbm,
              src_v, dst_v, msgs_v, out_v, shared, feat_sh, gsem):
    c = lax.axis_index("c")
    s = lax.axis_index("s")
    w = s * 2 + c
    base = w * E_PER_W
    pltpu.sync_copy(edges_hbm.at[pl.ds(base, E_PER_W)], src_v)
    pltpu.sync_copy(edges_hbm.at[pl.ds(E + base, E_PER_W)], dst_v)
    nslc = pl.ds(s * NODES_PER_TILE, NODES_PER_TILE)
    pltpu.sync_copy(zeros_hbm.at[nslc], shared.at[nslc])
    pltpu.sync_copy(feat_hbm.at[nslc], feat_sh.at[nslc])
    plsc.subcore_barrier()

    def g_start(b, carry):
        off = pl.multiple_of(b * 128, 128)
        pltpu.async_copy(feat_sh.at[src_v.at[pl.ds(off, 128)]], msgs_v.at[b],
                         gsem)
        return carry

    lax.fori_loop(0, NB_AGG, g_start, 0)
    pltpu.async_copy(feat_sh.at[src_v.at[pl.ds(NB_AGG * 128, TAIL_AGG)]],
                     msgs_v.at[NB_AGG, pl.ds(0, TAIL_AGG)], gsem)

    def g_wait(b, carry):
        off = pl.multiple_of(b * 128, 128)
        pltpu.make_async_copy(feat_sh.at[src_v.at[pl.ds(off, 128)]],
                              msgs_v.at[b], gsem).wait()
        return carry

    lax.fori_loop(0, NB_AGG, g_wait, 0)
    pltpu.make_async_copy(feat_sh.at[src_v.at[pl.ds(NB_AGG * 128, TAIL_AGG)]],
                          msgs_v.at[NB_AGG, pl.ds(0, TAIL_AGG)], gsem).wait()

    def scat(b, carry):
        off = pl.multiple_of(b * 128, 128)
        pltpu.async_copy(msgs_v.at[b], shared.at[dst_v.at[pl.ds(off, 128)]],
                         gsem, add=True)
        return carry

    lax.fori_loop(0, NB_AGG, scat, 0)
    pltpu.async_copy(msgs_v.at[NB_AGG, pl.ds(0, TAIL_AGG)],
                     shared.at[dst_v.at[pl.ds(NB_AGG * 128, TAIL_AGG)]], gsem,
                     add=True)

    def s_wait(b, carry):
        off = pl.multiple_of(b * 128, 128)
        pltpu.make_async_copy(msgs_v.at[b],
                              shared.at[dst_v.at[pl.ds(off, 128)]], gsem).wait()
        return carry

    lax.fori_loop(0, NB_AGG, s_wait, 0)
    pltpu.make_async_copy(msgs_v.at[NB_AGG, pl.ds(0, TAIL_AGG)],
                          shared.at[dst_v.at[pl.ds(NB_AGG * 128, TAIL_AGG)]],
                          gsem).wait()
    plsc.subcore_barrier()
    pltpu.sync_copy(shared.at[nslc], out_v)
    pltpu.sync_copy(out_v, rst_hbm.at[c, nslc])


_agg_call = functools.partial(
    pl.kernel,
    out_type=jax.ShapeDtypeStruct((2, NPAD, C8), jnp.float32),
    mesh=_mesh,
    compiler_params=_sc_params,
    scratch_types=[
        pltpu.VMEM((E_PER_W,), jnp.int32),
        pltpu.VMEM((E_PER_W,), jnp.int32),
        pltpu.VMEM((NB_AGG + 1, 128, C8), jnp.float32),
        pltpu.VMEM((NODES_PER_TILE, C8), jnp.float32),
        pltpu.VMEM_SHARED((NPAD, C8), jnp.float32),
        pltpu.VMEM_SHARED((NPAD, C8), jnp.float32),
        pltpu.SemaphoreType.DMA,
    ],
)(_agg_body)


_MM_BLK = 1024


def _mm_body(h_ref, w_ref, y_ref):
    y_ref[...] = jnp.dot(h_ref[...], w_ref[...],
                         preferred_element_type=jnp.float32)


def _norm_body(y_ref, od_ref, feat_ref):
    nsrc = lax.rsqrt(jnp.maximum(od_ref[...], 1.0))
    feat_ref[...] = y_ref[...] * nsrc


_FIN_BLK = 1000


def _fin_body(rst_ref, id_ref, b_ref, out_ref):
    ndst = lax.rsqrt(jnp.maximum(id_ref[...], 1.0))
    acc = (rst_ref[0] + rst_ref[1]) * ndst + b_ref[...]
    out_ref[...] = acc[:, :C_OUT]


def kernel(h, edge_index, W, b):
    e32 = edge_index.astype(jnp.int32).reshape(-1)
    W8 = jnp.pad(W, ((0, 0), (0, C8 - C_OUT)))
    b8 = jnp.pad(b, (0, C8 - C_OUT)).reshape(1, C8)
    zeros1 = jnp.zeros((NPAD,), jnp.float32)
    zeros8 = jnp.zeros((NPAD, C8), jnp.float32)

    degs = _deg_call(e32, zeros1)
    od = degs[:NPAD].reshape(NPAD, 1)
    idg = degs[NPAD:].reshape(NPAD, 1)

    y = pl.pallas_call(
        _mm_body,
        grid=(NPAD // _MM_BLK,),
        in_specs=[pl.BlockSpec((_MM_BLK, F_IN), lambda i: (i, 0)),
                  pl.BlockSpec((F_IN, C8), lambda i: (0, 0))],
        out_specs=pl.BlockSpec((_MM_BLK, C8), lambda i: (i, 0)),
        out_shape=jax.ShapeDtypeStruct((NPAD, C8), jnp.float32),
    )(h, W8)

    feat = pl.pallas_call(
        _norm_body,
        grid=(NPAD // _MM_BLK,),
        in_specs=[pl.BlockSpec((_MM_BLK, C8), lambda i: (i, 0)),
                  pl.BlockSpec((_MM_BLK, 1), lambda i: (i, 0))],
        out_specs=pl.BlockSpec((_MM_BLK, C8), lambda i: (i, 0)),
        out_shape=jax.ShapeDtypeStruct((NPAD, C8), jnp.float32),
    )(y, od)

    rst = _agg_call(e32, feat, zeros8)

    out = pl.pallas_call(
        _fin_body,
        grid=(N // _FIN_BLK,),
        in_specs=[pl.BlockSpec((2, _FIN_BLK, C8), lambda i: (0, i, 0)),
                  pl.BlockSpec((_FIN_BLK, 1), lambda i: (i, 0)),
                  pl.BlockSpec((1, C8), lambda i: (0, 0))],
        out_specs=pl.BlockSpec((_FIN_BLK, C_OUT), lambda i: (i, 0)),
        out_shape=jax.ShapeDtypeStruct((N, C_OUT), jnp.float32),
    )(rst, idg, b8)
    return out


# speedup vs baseline: 34.0027x; 1.2741x over previous
import functools

import jax
import jax.numpy as jnp
from jax import lax
from jax.experimental import pallas as pl
from jax.experimental.pallas import tpu as pltpu
from jax.experimental.pallas import tpu_sc as plsc

N = 10000
NPAD = 10240
E = 320000
F_IN = 128
C_OUT = 7
C8 = 8

E_PER_TILE = E // 16
NB_DEG = E_PER_TILE // 128
TAIL_DEG = E_PER_TILE - NB_DEG * 128
E_PER_W = E // 32
NB_AGG = E_PER_W // 128
TAIL_AGG = E_PER_W - NB_AGG * 128
NODES_PER_TILE = NPAD // 16
EXP_PER_TILE = NODES_PER_TILE * C8

_mesh = plsc.VectorSubcoreMesh(core_axis_name="c", subcore_axis_name="s")
_sc_params = pltpu.CompilerParams(use_tc_tiling_on_sc=False)
_sc_params_nl = pltpu.CompilerParams(use_tc_tiling_on_sc=False,
                                     needs_layout_passes=False)


def _deg_body(edges_hbm, zeros_hbm, idxmap_hbm, nexp_hbm,
              idx_v, ones_v, deg_v, exp_v, im_v, tmp_v, shared, ssem):
    c = lax.axis_index("c")
    s = lax.axis_index("s")
    for i in range(8):
        ones_v[pl.ds(i * 16, 16)] = jnp.ones((16,), jnp.float32)
    nslc = pl.ds(s * NODES_PER_TILE, NODES_PER_TILE)
    pltpu.sync_copy(zeros_hbm.at[nslc], shared.at[nslc])
    pltpu.sync_copy(idxmap_hbm.at[...], im_v)
    pltpu.sync_copy(edges_hbm.at[pl.ds(c * E + s * E_PER_TILE, E_PER_TILE)], idx_v)
    plsc.subcore_barrier()

    def scat(b, carry):
        off = pl.multiple_of(b * 128, 128)
        pltpu.async_copy(ones_v, shared.at[idx_v.at[pl.ds(off, 128)]], ssem,
                         add=True)
        return carry

    lax.fori_loop(0, NB_DEG, scat, 0)
    pltpu.async_copy(ones_v.at[pl.ds(0, TAIL_DEG)],
                     shared.at[idx_v.at[pl.ds(NB_DEG * 128, TAIL_DEG)]], ssem,
                     add=True)

    def drain(b, carry):
        off = pl.multiple_of(b * 128, 128)
        pltpu.make_async_copy(ones_v, shared.at[idx_v.at[pl.ds(off, 128)]],
                              ssem).wait()
        return carry

    lax.fori_loop(0, NB_DEG, drain, 0)
    pltpu.make_async_copy(ones_v.at[pl.ds(0, TAIL_DEG)],
                          shared.at[idx_v.at[pl.ds(NB_DEG * 128, TAIL_DEG)]],
                          ssem).wait()
    plsc.subcore_barrier()
    pltpu.sync_copy(shared.at[nslc], deg_v)

    def newton(i, carry):
        x = jnp.maximum(deg_v[pl.ds(i * 16, 16)], 1.0)
        bits = plsc.bitcast(x, jnp.int32)
        yb = 0x5F3759DF - lax.shift_right_logical(bits, 1)
        y = plsc.bitcast(yb, jnp.float32)
        for _ in range(3):
            y = y * (1.5 - 0.5 * x * y * y)
        tmp_v[...] = y
        for j in range(8):
            exp_v[pl.ds(i * 128 + j * 16, 16)] = plsc.load_gather(
                tmp_v, [im_v[j, :]])
        return carry

    lax.fori_loop(0, NODES_PER_TILE // 16, newton, 0)
    pltpu.sync_copy(exp_v,
                    nexp_hbm.at[pl.ds((c * 16 + s) * EXP_PER_TILE,
                                      EXP_PER_TILE)])


_deg_call = functools.partial(
    pl.kernel,
    out_type=jax.ShapeDtypeStruct((2 * NPAD * C8,), jnp.float32),
    mesh=_mesh,
    compiler_params=_sc_params_nl,
    scratch_types=[
        pltpu.VMEM((E_PER_TILE,), jnp.int32),
        pltpu.VMEM((128,), jnp.float32),
        pltpu.VMEM((NODES_PER_TILE,), jnp.float32),
        pltpu.VMEM((EXP_PER_TILE,), jnp.float32),
        pltpu.VMEM((8, 16), jnp.int32),
        pltpu.VMEM((16,), jnp.float32),
        pltpu.VMEM_SHARED((NPAD,), jnp.float32),
        pltpu.SemaphoreType.DMA,
    ],
)(_deg_body)


def _agg_body(edges_hbm, feat_hbm, zeros_hbm, rst_hbm,
              src_v, dst_v, msgs_v, out_v, shared, feat_sh, gsem):
    c = lax.axis_index("c")
    s = lax.axis_index("s")
    w = s * 2 + c
    base = w * E_PER_W
    pltpu.sync_copy(edges_hbm.at[pl.ds(base, E_PER_W)], src_v)
    pltpu.sync_copy(edges_hbm.at[pl.ds(E + base, E_PER_W)], dst_v)
    nslc = pl.ds(s * NODES_PER_TILE, NODES_PER_TILE)
    pltpu.sync_copy(zeros_hbm.at[nslc], shared.at[nslc])
    pltpu.sync_copy(feat_hbm.at[nslc], feat_sh.at[nslc])
    plsc.subcore_barrier()

    def g_start(b, carry):
        off = pl.multiple_of(b * 128, 128)
        pltpu.async_copy(feat_sh.at[src_v.at[pl.ds(off, 128)]], msgs_v.at[b],
                         gsem)
        return carry

    lax.fori_loop(0, NB_AGG, g_start, 0)
    pltpu.async_copy(feat_sh.at[src_v.at[pl.ds(NB_AGG * 128, TAIL_AGG)]],
                     msgs_v.at[NB_AGG, pl.ds(0, TAIL_AGG)], gsem)

    def g_wait(b, carry):
        off = pl.multiple_of(b * 128, 128)
        pltpu.make_async_copy(feat_sh.at[src_v.at[pl.ds(off, 128)]],
                              msgs_v.at[b], gsem).wait()
        return carry

    lax.fori_loop(0, NB_AGG, g_wait, 0)
    pltpu.make_async_copy(feat_sh.at[src_v.at[pl.ds(NB_AGG * 128, TAIL_AGG)]],
                          msgs_v.at[NB_AGG, pl.ds(0, TAIL_AGG)], gsem).wait()

    def scat(b, carry):
        off = pl.multiple_of(b * 128, 128)
        pltpu.async_copy(msgs_v.at[b], shared.at[dst_v.at[pl.ds(off, 128)]],
                         gsem, add=True)
        return carry

    lax.fori_loop(0, NB_AGG, scat, 0)
    pltpu.async_copy(msgs_v.at[NB_AGG, pl.ds(0, TAIL_AGG)],
                     shared.at[dst_v.at[pl.ds(NB_AGG * 128, TAIL_AGG)]], gsem,
                     add=True)

    def s_wait(b, carry):
        off = pl.multiple_of(b * 128, 128)
        pltpu.make_async_copy(msgs_v.at[b],
                              shared.at[dst_v.at[pl.ds(off, 128)]], gsem).wait()
        return carry

    lax.fori_loop(0, NB_AGG, s_wait, 0)
    pltpu.make_async_copy(msgs_v.at[NB_AGG, pl.ds(0, TAIL_AGG)],
                          shared.at[dst_v.at[pl.ds(NB_AGG * 128, TAIL_AGG)]],
                          gsem).wait()
    plsc.subcore_barrier()
    pltpu.sync_copy(shared.at[nslc], out_v)
    pltpu.sync_copy(out_v, rst_hbm.at[c, nslc])


_agg_call = functools.partial(
    pl.kernel,
    out_type=jax.ShapeDtypeStruct((2, NPAD, C8), jnp.float32),
    mesh=_mesh,
    compiler_params=_sc_params,
    scratch_types=[
        pltpu.VMEM((E_PER_W,), jnp.int32),
        pltpu.VMEM((E_PER_W,), jnp.int32),
        pltpu.VMEM((NB_AGG + 1, 128, C8), jnp.float32),
        pltpu.VMEM((NODES_PER_TILE, C8), jnp.float32),
        pltpu.VMEM_SHARED((NPAD, C8), jnp.float32),
        pltpu.VMEM_SHARED((NPAD, C8), jnp.float32),
        pltpu.SemaphoreType.DMA,
    ],
)(_agg_body)


_MM_BLK = 128
FLAT_ROWS = NPAD * C8 // 128


def _mm_body(h_ref, w_ref, y_ref):
    y = lax.dot_general(h_ref[...], w_ref[...], (((2,), (0,)), ((), ())),
                        preferred_element_type=jnp.float32)
    y_ref[...] = y.reshape(_MM_BLK, 128)


def _norm_body(y_ref, ns_ref, feat_ref):
    feat_ref[...] = y_ref[...] * ns_ref[...]


def _fin_body(rst_ref, nd_ref, b_ref, out_ref):
    out_ref[...] = (rst_ref[0] + rst_ref[1]) * nd_ref[...] + b_ref[...]


def kernel(h, edge_index, W, b):
    e32 = edge_index.astype(jnp.int32).reshape(-1)
    W8 = jnp.pad(W, ((0, 0), (0, C8 - C_OUT)))
    b8 = jnp.pad(b, (0, C8 - C_OUT))
    bexp = jnp.tile(b8, 16).reshape(1, 128)
    zeros1 = jnp.zeros((NPAD,), jnp.float32)
    zeros8 = jnp.zeros((NPAD, C8), jnp.float32)
    idxmap = (jnp.arange(128, dtype=jnp.int32) // 8).reshape(8, 16)
    h3 = h.reshape(N // 16, 16, F_IN)

    nexp = _deg_call(e32, zeros1, idxmap)
    nsrc = nexp[:NPAD * C8].reshape(FLAT_ROWS, 128)
    ndst = nexp[NPAD * C8:].reshape(FLAT_ROWS, 128)

    y = pl.pallas_call(
        _mm_body,
        grid=(FLAT_ROWS // _MM_BLK,),
        in_specs=[pl.BlockSpec((_MM_BLK, 16, F_IN), lambda i: (i, 0, 0)),
                  pl.BlockSpec((F_IN, C8), lambda i: (0, 0))],
        out_specs=pl.BlockSpec((_MM_BLK, 128), lambda i: (i, 0)),
        out_shape=jax.ShapeDtypeStruct((FLAT_ROWS, 128), jnp.float32),
    )(h3, W8)

    feat = pl.pallas_call(
        _norm_body,
        grid=(FLAT_ROWS // _MM_BLK,),
        in_specs=[pl.BlockSpec((_MM_BLK, 128), lambda i: (i, 0)),
                  pl.BlockSpec((_MM_BLK, 128), lambda i: (i, 0))],
        out_specs=pl.BlockSpec((_MM_BLK, 128), lambda i: (i, 0)),
        out_shape=jax.ShapeDtypeStruct((FLAT_ROWS, 128), jnp.float32),
    )(y, nsrc)

    rst = _agg_call(e32, feat.reshape(NPAD, C8), zeros8)

    out640 = pl.pallas_call(
        _fin_body,
        grid=(FLAT_ROWS // _MM_BLK,),
        in_specs=[pl.BlockSpec((2, _MM_BLK, 128), lambda i: (0, i, 0)),
                  pl.BlockSpec((_MM_BLK, 128), lambda i: (i, 0)),
                  pl.BlockSpec((1, 128), lambda i: (0, 0))],
        out_specs=pl.BlockSpec((_MM_BLK, 128), lambda i: (i, 0)),
        out_shape=jax.ShapeDtypeStruct((FLAT_ROWS, 128), jnp.float32),
    )(rst.reshape(2, FLAT_ROWS, 128), ndst, bexp)
    return out640.reshape(NPAD, C8)[:N, :C_OUT]
